# baseline (speedup 1.0000x reference)
import jax
import jax.numpy as jnp
from jax.experimental import pallas as pl


_R_BLK = 1024


def _add_kernel(x_ref, e_ref, o_ref):
    o_ref[...] = x_ref[...] + e_ref[...]


def kernel(inputs, embeddings):
    batch, seq_len, dim = inputs.shape
    pos = embeddings[:seq_len]
    x2 = inputs.reshape(batch * seq_len, dim)
    nr = (batch * seq_len) // _R_BLK
    ne = seq_len // _R_BLK
    out = pl.pallas_call(
        _add_kernel,
        grid=(nr,),
        in_specs=[
            pl.BlockSpec((_R_BLK, dim), lambda i: (i, 0)),
            pl.BlockSpec((_R_BLK, dim), lambda i: (i % ne, 0)),
        ],
        out_specs=pl.BlockSpec((_R_BLK, dim), lambda i: (i, 0)),
        out_shape=jax.ShapeDtypeStruct(x2.shape, x2.dtype),
    )(x2, pos)
    return out.reshape(inputs.shape)


# speedup vs baseline: 1.2753x; 1.2753x over previous
import jax
import jax.numpy as jnp
from jax.experimental import pallas as pl


_S_BLK = 1024


def _add_kernel(x_ref, e_ref, o_ref):
    o_ref[...] = x_ref[...] + e_ref[...]


def kernel(inputs, embeddings):
    batch, seq_len, dim = inputs.shape
    pos = embeddings[:seq_len]
    ns = seq_len // _S_BLK
    return pl.pallas_call(
        _add_kernel,
        grid=(ns, batch),
        in_specs=[
            pl.BlockSpec((1, _S_BLK, dim), lambda s, b: (b, s, 0)),
            pl.BlockSpec((_S_BLK, dim), lambda s, b: (s, 0)),
        ],
        out_specs=pl.BlockSpec((1, _S_BLK, dim), lambda s, b: (b, s, 0)),
        out_shape=jax.ShapeDtypeStruct(inputs.shape, inputs.dtype),
    )(inputs, pos)


# speedup vs baseline: 1.3311x; 1.0438x over previous
import jax
import jax.numpy as jnp
from jax.experimental import pallas as pl


_S_BLK = 2048


def _add_kernel(x_ref, e_ref, o_ref):
    o_ref[...] = x_ref[...] + e_ref[...]


def kernel(inputs, embeddings):
    batch, seq_len, dim = inputs.shape
    pos = embeddings[:seq_len]
    ns = seq_len // _S_BLK
    return pl.pallas_call(
        _add_kernel,
        grid=(ns, batch),
        in_specs=[
            pl.BlockSpec((1, _S_BLK, dim), lambda s, b: (b, s, 0)),
            pl.BlockSpec((_S_BLK, dim), lambda s, b: (s, 0)),
        ],
        out_specs=pl.BlockSpec((1, _S_BLK, dim), lambda s, b: (b, s, 0)),
        out_shape=jax.ShapeDtypeStruct(inputs.shape, inputs.dtype),
    )(inputs, pos)
